# Initial kernel scaffold; baseline (speedup 1.0000x reference)
import jax
import jax.numpy as jnp
from jax.experimental import pallas as pl


def kernel(batch_data, table):
    raise NotImplementedError("write your pallas kernel here")


# speedup vs baseline: 6.2689x; 6.2689x over previous
import functools

import jax
import jax.numpy as jnp
from jax import lax
from jax.experimental import pallas as pl
from jax.experimental.pallas import tpu as pltpu
from jax.experimental.pallas import tpu_sc as plsc

_VOCAB = 5000
_D = 16
_BATCH = 16384
_HIST = 200
_B = _BATCH * _HIST
_NW = 32
_ROWS_PER_STREAM = 128
_CHUNK = 2048
_SUB = _CHUNK // _ROWS_PER_STREAM
_PER_W = _B // _NW
_STEPS = _PER_W // _CHUNK
_IDX_ROWS_PER_W = _PER_W // _ROWS_PER_STREAM


def _sc_gather_body(table_hbm, idx_hbm, out_hbm, idx_v, rows_v, sem):
    wid = lax.axis_index("s") * 2 + lax.axis_index("c")
    idx_row0 = wid * _IDX_ROWS_PER_W
    out_base = wid * _PER_W

    def step(i, carry):
        pltpu.sync_copy(idx_hbm.at[pl.ds(idx_row0 + i * _SUB, _SUB)], idx_v)
        copies = []
        for j in range(_SUB):
            copies.append(pltpu.async_copy(
                table_hbm.at[idx_v.at[j]],
                rows_v.at[pl.ds(j * _ROWS_PER_STREAM, _ROWS_PER_STREAM)],
                sem))
        for c in copies:
            c.wait()
        pltpu.sync_copy(rows_v, out_hbm.at[pl.ds(out_base + i * _CHUNK, _CHUNK)])
        return carry

    lax.fori_loop(0, _STEPS, step, 0)


@functools.cache
def _sc_gather():
    return pl.kernel(
        _sc_gather_body,
        out_type=jax.ShapeDtypeStruct((_B, _D), jnp.float32),
        mesh=plsc.VectorSubcoreMesh(core_axis_name="c", subcore_axis_name="s"),
        scratch_types=[
            pltpu.VMEM((_SUB, _ROWS_PER_STREAM), jnp.int32),
            pltpu.VMEM((_CHUNK, _D), jnp.float32),
            pltpu.SemaphoreType.DMA,
        ],
        compiler_params=pltpu.CompilerParams(use_tc_tiling_on_sc=False),
    )


def kernel(batch_data, table):
    idx = batch_data.astype(jnp.int32).reshape(_B // _ROWS_PER_STREAM,
                                               _ROWS_PER_STREAM)
    out = _sc_gather()(table, idx)
    return out.reshape(_BATCH, _HIST, _D)
